# Initial kernel scaffold; baseline (speedup 1.0000x reference)
import functools

import jax
import jax.numpy as jnp
from jax import lax
from jax.experimental import pallas as pl
from jax.experimental.pallas import tpu as pltpu
from jax.experimental.pallas import tpu_sc as plsc

_N_USERS = 25000
_N_ITEMS = 25000
_D = 64
_E = 800000
_HALF = 25000
_ACC_R = 25088
_PN = 2 * _ACC_R
_ROWS_PER_TILE = _ACC_R // 16
_C = 128
_EP = 800768
_NCHUNK = _EP // (16 * _C)
_SPREAD_MASK = 16383

_MESH = plsc.VectorSubcoreMesh(core_axis_name="c", subcore_axis_name="s")


def _prop_body(ego, rowa, cola, wa, out, colv, rowv, wv, rows, acc, sem):
    c = lax.axis_index("c")
    s = lax.axis_index("s")
    base = c * _HALF

    def _zrow(i, carry):
        for k in range(4):
            rows[i, pl.ds(k * 16, 16)] = jnp.zeros((16,), jnp.float32)
        return carry

    lax.fori_loop(0, _C, _zrow, 0)
    zbase = s * _ROWS_PER_TILE
    for j in range(12):
        pltpu.sync_copy(rows, acc.at[pl.ds(zbase + j * _C, _C)])
    pltpu.sync_copy(rows.at[pl.ds(0, 32)], acc.at[pl.ds(zbase + 12 * _C, 32)])
    plsc.subcore_barrier()

    lanes = jnp.arange(16, dtype=jnp.int32)

    def _chunk(i, carry):
        e0 = (i * 16 + s) * _C
        ca = pltpu.async_copy(cola.at[pl.ds(e0, _C)], colv, sem)
        ra = pltpu.async_copy(rowa.at[pl.ds(e0, _C)], rowv, sem)
        wcp = pltpu.async_copy(wa.at[pl.ds(e0, _C)], wv, sem)
        ca.wait()
        ra.wait()
        wcp.wait()
        for g in range(_C // 16):
            sl = pl.ds(g * 16, 16)
            cv = colv[sl]
            colv[sl] = jnp.where(cv >= _HALF, cv + (_ACC_R - _HALF), cv)
            rv = rowv[sl]
            inh = (rv >= base) & (rv < base + _HALF)
            spread = (lanes + (e0 + g * 16)) & _SPREAD_MASK
            rowv[sl] = jnp.where(inh, rv - base, spread)
            wvec = wv[sl]
            wv[sl] = jnp.where(inh, wvec, jnp.zeros((16,), jnp.float32))
        pltpu.async_copy(ego.at[colv], rows, sem).wait()

        def _mul(e, mc):
            ws = wv[e]
            for k in range(4):
                sl2 = pl.ds(k * 16, 16)
                rows[e, sl2] = rows[e, sl2] * ws
            return mc

        lax.fori_loop(0, _C, _mul, 0)
        pltpu.sync_copy(rows, acc.at[rowv], add=True)
        return carry

    lax.fori_loop(0, _NCHUNK, _chunk, 0)
    plsc.subcore_barrier()

    wb = s * _ROWS_PER_TILE
    for j in range(12):
        pltpu.sync_copy(acc.at[pl.ds(wb + j * _C, _C)],
                        out.at[pl.ds(c * _ACC_R + wb + j * _C, _C)])
    pltpu.sync_copy(acc.at[pl.ds(wb + 12 * _C, 32)],
                    out.at[pl.ds(c * _ACC_R + wb + 12 * _C, 32)])


_propagate = functools.partial(
    pl.kernel,
    out_type=jax.ShapeDtypeStruct((_PN, _D), jnp.float32),
    mesh=_MESH,
    scratch_types=[
        pltpu.VMEM((_C,), jnp.int32),
        pltpu.VMEM((_C,), jnp.int32),
        pltpu.VMEM((_C,), jnp.float32),
        pltpu.VMEM((_C, _D), jnp.float32),
        pltpu.VMEM_SHARED((_ACC_R, _D), jnp.float32),
        pltpu.SemaphoreType.DMA,
    ],
)(_prop_body)


def _mean_body(a, b, c, d, o):
    o[...] = (a[...] + b[...] + c[...] + d[...]) * 0.25


_mean4 = pl.pallas_call(
    _mean_body,
    grid=(_PN // 1568,),
    in_specs=[pl.BlockSpec((1568, _D), lambda i: (i, 0))] * 4,
    out_specs=pl.BlockSpec((1568, _D), lambda i: (i, 0)),
    out_shape=jax.ShapeDtypeStruct((_PN, _D), jnp.float32),
)


def kernel(user_emb, item_emb, edge_weight, edge_index):
    row = edge_index[0]
    col = edge_index[1]
    npad = _EP - _E
    rowp = jnp.concatenate(
        [row, (jnp.arange(npad, dtype=jnp.int32) * 37) & _SPREAD_MASK])
    colp = jnp.concatenate([col, jnp.zeros((npad,), jnp.int32)])
    wp = jnp.concatenate([edge_weight, jnp.zeros((npad,), jnp.float32)])
    zpad = jnp.zeros((_ACC_R - _HALF, _D), jnp.float32)
    ego0 = jnp.concatenate([user_emb, zpad, item_emb, zpad], axis=0)
    e1 = _propagate(ego0, rowp, colp, wp)
    e2 = _propagate(e1, rowp, colp, wp)
    e3 = _propagate(e2, rowp, colp, wp)
    final = _mean4(ego0, e1, e2, e3)
    return final[:_N_USERS], final[_ACC_R:_ACC_R + _N_ITEMS]


# speedup vs baseline: 2.4159x; 2.4159x over previous
import functools

import jax
import jax.numpy as jnp
from jax import lax
from jax.experimental import pallas as pl
from jax.experimental.pallas import tpu as pltpu
from jax.experimental.pallas import tpu_sc as plsc

_N_USERS = 25000
_N_ITEMS = 25000
_D = 64
_E = 800000
_HALF = 25000
_ACC_R = 25600
_PN = 2 * _ACC_R
_ROWS_PER_TILE = _ACC_R // 16
_C = 128
_EP = 800768
_NCHUNK = _EP // (16 * _C)
_DUMP_MASK = 511

_MESH = plsc.VectorSubcoreMesh(core_axis_name="c", subcore_axis_name="s")


def _prop_body(ego, rowa, cola, wa, out, colv, rowv, wv, rows, acc, sem):
    c = lax.axis_index("c")
    s = lax.axis_index("s")
    base = c * _HALF

    def _zrow(i, carry):
        for k in range(4):
            rows[i, pl.ds(k * 16, 16)] = jnp.zeros((16,), jnp.float32)
        return carry

    lax.fori_loop(0, _C, _zrow, 0)
    zbase = s * _ROWS_PER_TILE
    for j in range(12):
        pltpu.sync_copy(rows, acc.at[pl.ds(zbase + j * _C, _C)])
    pltpu.sync_copy(rows.at[pl.ds(0, 64)], acc.at[pl.ds(zbase + 12 * _C, 64)])
    plsc.subcore_barrier()

    lanes = jnp.arange(16, dtype=jnp.int32)

    def _chunk(i, carry):
        e0 = (i * 16 + s) * _C
        ca = pltpu.async_copy(cola.at[pl.ds(e0, _C)], colv, sem)
        ra = pltpu.async_copy(rowa.at[pl.ds(e0, _C)], rowv, sem)
        wcp = pltpu.async_copy(wa.at[pl.ds(e0, _C)], wv, sem)
        ca.wait()
        ra.wait()
        wcp.wait()
        for g in range(_C // 16):
            sl = pl.ds(g * 16, 16)
            cv = colv[sl]
            colv[sl] = jnp.where(cv >= _HALF, cv + (_ACC_R - _HALF), cv)
            rv = rowv[sl]
            inh = (rv >= base) & (rv < base + _HALF)
            spread = _HALF + ((lanes + (e0 + g * 16)) & _DUMP_MASK)
            rowv[sl] = jnp.where(inh, rv - base, spread)
            wvec = wv[sl]
            wv[sl] = jnp.where(inh, wvec, jnp.zeros((16,), jnp.float32))
        pltpu.async_copy(ego.at[colv], rows, sem).wait()

        def _mul(g, mc):
            wvec = wv[pl.ds(g * 16, 16)]
            for j in range(16):
                ws = wvec[j]
                e = g * 16 + j
                for k in range(4):
                    sl2 = pl.ds(k * 16, 16)
                    rows[e, sl2] = rows[e, sl2] * ws
            return mc

        lax.fori_loop(0, _C // 16, _mul, 0)
        pltpu.sync_copy(rows, acc.at[rowv], add=True)
        return carry

    lax.fori_loop(0, _NCHUNK, _chunk, 0)
    plsc.subcore_barrier()

    wb = s * _ROWS_PER_TILE
    for j in range(12):
        pltpu.sync_copy(acc.at[pl.ds(wb + j * _C, _C)],
                        out.at[pl.ds(c * _ACC_R + wb + j * _C, _C)])
    pltpu.sync_copy(acc.at[pl.ds(wb + 12 * _C, 64)],
                    out.at[pl.ds(c * _ACC_R + wb + 12 * _C, 64)])


_propagate = functools.partial(
    pl.kernel,
    out_type=jax.ShapeDtypeStruct((_PN, _D), jnp.float32),
    mesh=_MESH,
    scratch_types=[
        pltpu.VMEM((_C,), jnp.int32),
        pltpu.VMEM((_C,), jnp.int32),
        pltpu.VMEM((_C,), jnp.float32),
        pltpu.VMEM((_C, _D), jnp.float32),
        pltpu.VMEM_SHARED((_ACC_R, _D), jnp.float32),
        pltpu.SemaphoreType.DMA,
    ],
    compiler_params=pltpu.CompilerParams(use_tc_tiling_on_sc=False),
)(_prop_body)


def _mean_body(a, b, c, d, o):
    o[...] = (a[...] + b[...] + c[...] + d[...]) * 0.25


_mean4 = pl.pallas_call(
    _mean_body,
    grid=(_PN // 1600,),
    in_specs=[pl.BlockSpec((1600, _D), lambda i: (i, 0))] * 4,
    out_specs=pl.BlockSpec((1600, _D), lambda i: (i, 0)),
    out_shape=jax.ShapeDtypeStruct((_PN, _D), jnp.float32),
)


def kernel(user_emb, item_emb, edge_weight, edge_index):
    row = edge_index[0]
    col = edge_index[1]
    npad = _EP - _E
    rowp = jnp.concatenate(
        [row, jnp.full((npad,), -1, jnp.int32)])
    colp = jnp.concatenate([col, jnp.zeros((npad,), jnp.int32)])
    wp = jnp.concatenate([edge_weight, jnp.zeros((npad,), jnp.float32)])
    zpad = jnp.zeros((_ACC_R - _HALF, _D), jnp.float32)
    ego0 = jnp.concatenate([user_emb, zpad, item_emb, zpad], axis=0)
    e1 = _propagate(ego0, rowp, colp, wp)
    e2 = _propagate(e1, rowp, colp, wp)
    e3 = _propagate(e2, rowp, colp, wp)
    final = _mean4(ego0, e1, e2, e3)
    return final[:_N_USERS], final[_ACC_R:_ACC_R + _N_ITEMS]


# speedup vs baseline: 7.3577x; 3.0455x over previous
import functools

import jax
import jax.numpy as jnp
from jax import lax
from jax.experimental import pallas as pl
from jax.experimental.pallas import tpu as pltpu
from jax.experimental.pallas import tpu_sc as plsc

_N_USERS = 25000
_N_ITEMS = 25000
_D = 64
_E = 800000
_HALF = 25000
_ACC_R = 25600
_PN = 2 * _ACC_R
_ROWS_PER_TILE = _ACC_R // 16
_C = 128
_T = 196
_EPROC = 2 * _T * 16 * _C
_EARR = ((2 * _T + 1) * 16 + 15) * _C + _C

_MESH = plsc.VectorSubcoreMesh(core_axis_name="c", subcore_axis_name="s")


def _prop_body(ego, rowa, cola, wa, out,
               colv0, colv1, rowv0, rowv1, wv0, wv1, wm0, wm1,
               sidx0, sidx1, rows0, rows1,
               semi0, semi1, semg0, semg1, sems0, sems1, acc):
    c = lax.axis_index("c")
    s = lax.axis_index("s")
    base = c * _HALF
    colv = (colv0, colv1)
    rowv = (rowv0, rowv1)
    wv = (wv0, wv1)
    wm = (wm0, wm1)
    sidx = (sidx0, sidx1)
    rows = (rows0, rows1)
    semi = (semi0, semi1)
    semg = (semg0, semg1)
    sems = (sems0, sems1)
    lanes = jnp.arange(16, dtype=jnp.int32)

    def idx_start(p, ci):
        e0 = (ci * 16 + s) * _C
        pltpu.async_copy(cola.at[pl.ds(e0, _C)], colv[p], semi[p])
        pltpu.async_copy(rowa.at[pl.ds(e0, _C)], rowv[p], semi[p])
        pltpu.async_copy(wa.at[pl.ds(e0, _C)], wv[p], semi[p])

    def idx_wait(p):
        pltpu.make_async_copy(cola.at[pl.ds(0, _C)], colv[p], semi[p]).wait()
        pltpu.make_async_copy(rowa.at[pl.ds(0, _C)], rowv[p], semi[p]).wait()
        pltpu.make_async_copy(wa.at[pl.ds(0, _C)], wv[p], semi[p]).wait()

    def prep(p):
        for g in range(_C // 16):
            sl = pl.ds(g * 16, 16)
            cv = colv[p][sl]
            colv[p][sl] = jnp.where(cv >= _HALF, cv + (_ACC_R - _HALF), cv)
            rv = rowv[p][sl]
            inh = (rv >= base) & (rv < base + _HALF)
            sidx[p][sl] = jnp.where(inh, rv - base, _HALF + g * 16 + lanes)
            wm[p][sl] = jnp.where(inh, wv[p][sl],
                                  jnp.zeros((16,), jnp.float32))

    def gather_start(p):
        pltpu.async_copy(ego.at[colv[p]], rows[p], semg[p])

    def gather_wait(p):
        pltpu.make_async_copy(ego.at[pl.ds(0, _C)], rows[p], semg[p]).wait()

    def mul(p):
        def _mg(g, mc):
            wvec = wm[p][pl.ds(g * 16, 16)]
            for j in range(16):
                ws = wvec[j]
                e = g * 16 + j
                for k in range(4):
                    sl2 = pl.ds(k * 16, 16)
                    rows[p][e, sl2] = rows[p][e, sl2] * ws
            return mc

        lax.fori_loop(0, _C // 16, _mg, 0)

    def scatter_start(p):
        pltpu.async_copy(rows[p], acc.at[sidx[p]], sems[p], add=True)

    def scatter_wait(p):
        pltpu.make_async_copy(rows[p], acc.at[pl.ds(0, _C)], sems[p]).wait()

    def _zrow(i, carry):
        for k in range(4):
            rows0[i, pl.ds(k * 16, 16)] = jnp.zeros((16,), jnp.float32)
        return carry

    lax.fori_loop(0, _C, _zrow, 0)
    zbase = s * _ROWS_PER_TILE
    for j in range(12):
        pltpu.sync_copy(rows0, acc.at[pl.ds(zbase + j * _C, _C)])
    pltpu.sync_copy(rows0.at[pl.ds(0, 64)], acc.at[pl.ds(zbase + 12 * _C, 64)])
    plsc.subcore_barrier()

    idx_start(0, 0)
    idx_wait(0)
    prep(0)
    gather_start(0)
    idx_start(1, 1)

    def _iter(t, carry):
        idx_wait(1)

        @pl.when(t > 0)
        def _():
            scatter_wait(1)

        prep(1)
        gather_start(1)
        gather_wait(0)
        idx_start(0, 2 * t + 2)
        mul(0)
        scatter_start(0)
        gather_wait(1)
        idx_start(1, 2 * t + 3)
        mul(1)
        idx_wait(0)
        scatter_wait(0)
        prep(0)
        gather_start(0)
        scatter_start(1)
        return carry

    lax.fori_loop(0, _T, _iter, 0)
    gather_wait(0)
    idx_wait(1)
    scatter_wait(1)
    plsc.subcore_barrier()

    wb = s * _ROWS_PER_TILE
    for j in range(12):
        pltpu.sync_copy(acc.at[pl.ds(wb + j * _C, _C)],
                        out.at[pl.ds(c * _ACC_R + wb + j * _C, _C)])
    pltpu.sync_copy(acc.at[pl.ds(wb + 12 * _C, 64)],
                    out.at[pl.ds(c * _ACC_R + wb + 12 * _C, 64)])


def _scratch_types():
    i32 = jnp.int32
    f32 = jnp.float32
    return [
        pltpu.VMEM((_C,), i32), pltpu.VMEM((_C,), i32),
        pltpu.VMEM((_C,), i32), pltpu.VMEM((_C,), i32),
        pltpu.VMEM((_C,), f32), pltpu.VMEM((_C,), f32),
        pltpu.VMEM((_C,), f32), pltpu.VMEM((_C,), f32),
        pltpu.VMEM((_C,), i32), pltpu.VMEM((_C,), i32),
        pltpu.VMEM((_C, _D), f32), pltpu.VMEM((_C, _D), f32),
        pltpu.SemaphoreType.DMA, pltpu.SemaphoreType.DMA,
        pltpu.SemaphoreType.DMA, pltpu.SemaphoreType.DMA,
        pltpu.SemaphoreType.DMA, pltpu.SemaphoreType.DMA,
    ]


_propagate = functools.partial(
    pl.kernel,
    out_type=jax.ShapeDtypeStruct((_PN, _D), jnp.float32),
    mesh=_MESH,
    scratch_types=_scratch_types() + [
        pltpu.VMEM_SHARED((_ACC_R, _D), jnp.float32),
    ],
    compiler_params=pltpu.CompilerParams(use_tc_tiling_on_sc=False),
)(_prop_body)


def _mean_body(a, b, c, d, o):
    o[...] = (a[...] + b[...] + c[...] + d[...]) * 0.25


_mean4 = pl.pallas_call(
    _mean_body,
    grid=(_PN // 1600,),
    in_specs=[pl.BlockSpec((1600, _D), lambda i: (i, 0))] * 4,
    out_specs=pl.BlockSpec((1600, _D), lambda i: (i, 0)),
    out_shape=jax.ShapeDtypeStruct((_PN, _D), jnp.float32),
)


def kernel(user_emb, item_emb, edge_weight, edge_index):
    row = edge_index[0]
    col = edge_index[1]
    npad = _EARR - _E
    rowp = jnp.concatenate([row, jnp.full((npad,), -1, jnp.int32)])
    colp = jnp.concatenate(
        [col, (jnp.arange(npad, dtype=jnp.int32) * 7919) % (2 * _HALF)])
    wp = jnp.concatenate([edge_weight, jnp.zeros((npad,), jnp.float32)])
    zpad = jnp.zeros((_ACC_R - _HALF, _D), jnp.float32)
    ego0 = jnp.concatenate([user_emb, zpad, item_emb, zpad], axis=0)
    e1 = _propagate(ego0, rowp, colp, wp)
    e2 = _propagate(e1, rowp, colp, wp)
    e3 = _propagate(e2, rowp, colp, wp)
    final = _mean4(ego0, e1, e2, e3)
    return final[:_N_USERS], final[_ACC_R:_ACC_R + _N_ITEMS]


# speedup vs baseline: 7.3606x; 1.0004x over previous
import functools

import jax
import jax.numpy as jnp
from jax import lax
from jax.experimental import pallas as pl
from jax.experimental.pallas import tpu as pltpu
from jax.experimental.pallas import tpu_sc as plsc

_N_USERS = 25000
_N_ITEMS = 25000
_D = 64
_E = 800000
_HALF = 25000
_ACC_R = 25600
_PN = 2 * _ACC_R
_ROWS_PER_TILE = _ACC_R // 16
_C = 128
_T = 196
_EPROC = 2 * _T * 16 * _C
_EARR = ((2 * _T + 1) * 16 + 15) * _C + _C
_DUMP_MASK = 511

_MESH = plsc.VectorSubcoreMesh(core_axis_name="c", subcore_axis_name="s")


def _prop_body(ego, rowa, cola, wa, out,
               colv0, colv1, rowv0, rowv1, wv0, wv1, wm0, wm1,
               sidx0, sidx1, rows0, rows1,
               semi0, semi1, semg0, semg1, sems0, sems1, acc):
    c = lax.axis_index("c")
    s = lax.axis_index("s")
    base = c * _HALF
    colv = (colv0, colv1)
    rowv = (rowv0, rowv1)
    wv = (wv0, wv1)
    wm = (wm0, wm1)
    sidx = (sidx0, sidx1)
    rows = (rows0, rows1)
    semi = (semi0, semi1)
    semg = (semg0, semg1)
    sems = (sems0, sems1)
    lanes = jnp.arange(16, dtype=jnp.int32)

    def idx_start(p, ci):
        e0 = (ci * 16 + s) * _C
        pltpu.async_copy(cola.at[pl.ds(e0, _C)], colv[p], semi[p])
        pltpu.async_copy(rowa.at[pl.ds(e0, _C)], rowv[p], semi[p])
        pltpu.async_copy(wa.at[pl.ds(e0, _C)], wv[p], semi[p])

    def idx_wait(p):
        pltpu.make_async_copy(cola.at[pl.ds(0, _C)], colv[p], semi[p]).wait()
        pltpu.make_async_copy(rowa.at[pl.ds(0, _C)], rowv[p], semi[p]).wait()
        pltpu.make_async_copy(wa.at[pl.ds(0, _C)], wv[p], semi[p]).wait()

    def prep(p):
        for g in range(_C // 16):
            sl = pl.ds(g * 16, 16)
            cv = colv[p][sl]
            colv[p][sl] = jnp.where(cv >= _HALF, cv + (_ACC_R - _HALF), cv)
            rv = rowv[p][sl]
            inh = (rv >= base) & (rv < base + _HALF)
            spread = _HALF + ((lanes + g * 16) & _DUMP_MASK)
            sidx[p][sl] = jnp.where(inh, rv - base, spread)
            wm[p][sl] = jnp.where(inh, wv[p][sl],
                                  jnp.zeros((16,), jnp.float32))

    def gather_start(p):
        pltpu.async_copy(ego.at[colv[p]], rows[p], semg[p])

    def gather_wait(p):
        pltpu.make_async_copy(ego.at[pl.ds(0, _C)], rows[p], semg[p]).wait()

    def mul(p):
        def _mg(g, mc):
            wvec = wm[p][pl.ds(g * 16, 16)]
            for j in range(16):
                ws = wvec[j]
                e = g * 16 + j
                for k in range(4):
                    sl2 = pl.ds(k * 16, 16)
                    rows[p][e, sl2] = rows[p][e, sl2] * ws
            return mc

        lax.fori_loop(0, _C // 16, _mg, 0)

    def scatter_start(p):
        pltpu.async_copy(rows[p], acc.at[sidx[p]], sems[p], add=True)

    def scatter_wait(p):
        pltpu.make_async_copy(rows[p], acc.at[pl.ds(0, _C)], sems[p]).wait()

    def _zrow(i, carry):
        for k in range(4):
            rows0[i, pl.ds(k * 16, 16)] = jnp.zeros((16,), jnp.float32)
        return carry

    lax.fori_loop(0, _C, _zrow, 0)
    zbase = s * _ROWS_PER_TILE
    for j in range(12):
        pltpu.sync_copy(rows0, acc.at[pl.ds(zbase + j * _C, _C)])
    pltpu.sync_copy(rows0.at[pl.ds(0, 64)], acc.at[pl.ds(zbase + 12 * _C, 64)])
    plsc.subcore_barrier()

    idx_start(0, 0)
    idx_wait(0)
    prep(0)
    gather_start(0)
    idx_start(1, 1)

    def _iter(t, carry):
        idx_wait(1)

        @pl.when(t > 0)
        def _():
            scatter_wait(1)

        prep(1)
        gather_start(1)
        gather_wait(0)
        idx_start(0, 2 * t + 2)
        mul(0)
        scatter_start(0)
        gather_wait(1)
        idx_start(1, 2 * t + 3)
        mul(1)
        idx_wait(0)
        scatter_wait(0)
        prep(0)
        gather_start(0)
        scatter_start(1)
        return carry

    lax.fori_loop(0, _T, _iter, 0)
    gather_wait(0)
    idx_wait(1)
    scatter_wait(1)
    plsc.subcore_barrier()

    wb = s * _ROWS_PER_TILE
    for j in range(12):
        pltpu.sync_copy(acc.at[pl.ds(wb + j * _C, _C)],
                        out.at[pl.ds(c * _ACC_R + wb + j * _C, _C)])
    pltpu.sync_copy(acc.at[pl.ds(wb + 12 * _C, 64)],
                    out.at[pl.ds(c * _ACC_R + wb + 12 * _C, 64)])


_propagate = functools.partial(
    pl.kernel,
    out_type=jax.ShapeDtypeStruct((_PN, _D), jnp.float32),
    mesh=_MESH,
    scratch_types=[
        pltpu.VMEM((_C,), jnp.int32), pltpu.VMEM((_C,), jnp.int32),
        pltpu.VMEM((_C,), jnp.int32), pltpu.VMEM((_C,), jnp.int32),
        pltpu.VMEM((_C,), jnp.float32), pltpu.VMEM((_C,), jnp.float32),
        pltpu.VMEM((_C,), jnp.float32), pltpu.VMEM((_C,), jnp.float32),
        pltpu.VMEM((_C,), jnp.int32), pltpu.VMEM((_C,), jnp.int32),
        pltpu.VMEM((_C, _D), jnp.float32), pltpu.VMEM((_C, _D), jnp.float32),
        pltpu.SemaphoreType.DMA, pltpu.SemaphoreType.DMA,
        pltpu.SemaphoreType.DMA, pltpu.SemaphoreType.DMA,
        pltpu.SemaphoreType.DMA, pltpu.SemaphoreType.DMA,
        pltpu.VMEM_SHARED((_ACC_R, _D), jnp.float32),
    ],
    compiler_params=pltpu.CompilerParams(use_tc_tiling_on_sc=False),
)(_prop_body)


def _mean_body(a, b, c, d, o):
    o[...] = (a[...] + b[...] + c[...] + d[...]) * 0.25


_mean4 = pl.pallas_call(
    _mean_body,
    grid=(_PN // 1600,),
    in_specs=[pl.BlockSpec((1600, _D), lambda i: (i, 0))] * 4,
    out_specs=pl.BlockSpec((1600, _D), lambda i: (i, 0)),
    out_shape=jax.ShapeDtypeStruct((_PN, _D), jnp.float32),
)


def kernel(user_emb, item_emb, edge_weight, edge_index):
    row = edge_index[0]
    col = edge_index[1]
    npad = _EARR - _E
    rowp = jnp.concatenate([row, jnp.full((npad,), -1, jnp.int32)])
    colp = jnp.concatenate(
        [col, (jnp.arange(npad, dtype=jnp.int32) * 7919) % (2 * _HALF)])
    wp = jnp.concatenate([edge_weight, jnp.zeros((npad,), jnp.float32)])
    zpad = jnp.zeros((_ACC_R - _HALF, _D), jnp.float32)
    ego0 = jnp.concatenate([user_emb, zpad, item_emb, zpad], axis=0)
    e1 = _propagate(ego0, rowp, colp, wp)
    e2 = _propagate(e1, rowp, colp, wp)
    e3 = _propagate(e2, rowp, colp, wp)
    final = _mean4(ego0, e1, e2, e3)
    return final[:_N_USERS], final[_ACC_R:_ACC_R + _N_ITEMS]


# speedup vs baseline: 7.3721x; 1.0016x over previous
import functools

import jax
import jax.numpy as jnp
from jax import lax
from jax.experimental import pallas as pl
from jax.experimental.pallas import tpu as pltpu
from jax.experimental.pallas import tpu_sc as plsc

_N_USERS = 25000
_N_ITEMS = 25000
_D = 64
_E = 800000
_HALF = 25000
_ACC_R = 25600
_PN = 2 * _ACC_R
_ROWS_PER_TILE = _ACC_R // 16
_C = 128
_T = 196
_EPROC = 2 * _T * 16 * _C
_EARR = ((2 * _T + 1) * 16 + 15) * _C + _C
_DUMP_MASK = 511

_MESH = plsc.VectorSubcoreMesh(core_axis_name="c", subcore_axis_name="s")


def _prop_body(ego, rcw, wa, out,
               rcw0, rcw1, wv0, wv1, wm0, wm1,
               sidx0, sidx1, rows0, rows1,
               semi0, semi1, semg0, semg1, sems0, sems1, acc):
    c = lax.axis_index("c")
    s = lax.axis_index("s")
    base = c * _HALF
    rcwb = (rcw0, rcw1)
    wv = (wv0, wv1)
    wm = (wm0, wm1)
    sidx = (sidx0, sidx1)
    rows = (rows0, rows1)
    semi = (semi0, semi1)
    semg = (semg0, semg1)
    sems = (sems0, sems1)
    lanes = jnp.arange(16, dtype=jnp.int32)

    def idx_start(p, ci):
        gi = ci * 16 + s
        pltpu.async_copy(rcw.at[gi], rcwb[p], semi[p])
        pltpu.async_copy(wa.at[pl.ds(gi * _C, _C)], wv[p], semi[p])

    def idx_wait(p):
        pltpu.make_async_copy(rcw.at[0], rcwb[p], semi[p]).wait()
        pltpu.make_async_copy(wa.at[pl.ds(0, _C)], wv[p], semi[p]).wait()

    def prep(p):
        for g in range(_C // 16):
            sl = pl.ds(g * 16, 16)
            cv = rcwb[p][1, sl]
            rcwb[p][1, sl] = jnp.where(cv >= _HALF, cv + (_ACC_R - _HALF), cv)
            rv = rcwb[p][0, sl]
            inh = (rv >= base) & (rv < base + _HALF)
            spread = _HALF + ((lanes + g * 16) & _DUMP_MASK)
            sidx[p][sl] = jnp.where(inh, rv - base, spread)
            wm[p][sl] = jnp.where(inh, wv[p][sl],
                                  jnp.zeros((16,), jnp.float32))

    def gather_start(p):
        pltpu.async_copy(ego.at[rcwb[p].at[1]], rows[p], semg[p])

    def gather_wait(p):
        pltpu.make_async_copy(ego.at[pl.ds(0, _C)], rows[p], semg[p]).wait()

    def mul(p):
        def _mg(g, mc):
            wvec = wm[p][pl.ds(g * 16, 16)]
            for j in range(16):
                ws = wvec[j]
                e = g * 16 + j
                for k in range(4):
                    sl2 = pl.ds(k * 16, 16)
                    rows[p][e, sl2] = rows[p][e, sl2] * ws
            return mc

        lax.fori_loop(0, _C // 16, _mg, 0)

    def scatter_start(p):
        pltpu.async_copy(rows[p], acc.at[sidx[p]], sems[p], add=True)

    def scatter_wait(p):
        pltpu.make_async_copy(rows[p], acc.at[pl.ds(0, _C)], sems[p]).wait()

    def _zrow(i, carry):
        for k in range(4):
            rows0[i, pl.ds(k * 16, 16)] = jnp.zeros((16,), jnp.float32)
        return carry

    lax.fori_loop(0, _C, _zrow, 0)
    zbase = s * _ROWS_PER_TILE
    for j in range(12):
        pltpu.sync_copy(rows0, acc.at[pl.ds(zbase + j * _C, _C)])
    pltpu.sync_copy(rows0.at[pl.ds(0, 64)], acc.at[pl.ds(zbase + 12 * _C, 64)])
    plsc.subcore_barrier()

    idx_start(0, 0)
    idx_wait(0)
    prep(0)
    gather_start(0)
    idx_start(1, 1)

    def _iter(t, carry):
        idx_wait(1)

        @pl.when(t > 0)
        def _():
            scatter_wait(1)

        prep(1)
        gather_start(1)
        gather_wait(0)
        idx_start(0, 2 * t + 2)
        mul(0)
        scatter_start(0)
        gather_wait(1)
        idx_start(1, 2 * t + 3)
        mul(1)
        idx_wait(0)
        scatter_wait(0)
        prep(0)
        gather_start(0)
        scatter_start(1)
        return carry

    lax.fori_loop(0, _T, _iter, 0)
    gather_wait(0)
    idx_wait(1)
    scatter_wait(1)
    plsc.subcore_barrier()

    wb = s * _ROWS_PER_TILE
    for j in range(12):
        pltpu.sync_copy(acc.at[pl.ds(wb + j * _C, _C)],
                        out.at[pl.ds(c * _ACC_R + wb + j * _C, _C)])
    pltpu.sync_copy(acc.at[pl.ds(wb + 12 * _C, 64)],
                    out.at[pl.ds(c * _ACC_R + wb + 12 * _C, 64)])


_propagate = functools.partial(
    pl.kernel,
    out_type=jax.ShapeDtypeStruct((_PN, _D), jnp.float32),
    mesh=_MESH,
    scratch_types=[
        pltpu.VMEM((2, _C), jnp.int32), pltpu.VMEM((2, _C), jnp.int32),
        pltpu.VMEM((_C,), jnp.float32), pltpu.VMEM((_C,), jnp.float32),
        pltpu.VMEM((_C,), jnp.float32), pltpu.VMEM((_C,), jnp.float32),
        pltpu.VMEM((_C,), jnp.int32), pltpu.VMEM((_C,), jnp.int32),
        pltpu.VMEM((_C, _D), jnp.float32), pltpu.VMEM((_C, _D), jnp.float32),
        pltpu.SemaphoreType.DMA, pltpu.SemaphoreType.DMA,
        pltpu.SemaphoreType.DMA, pltpu.SemaphoreType.DMA,
        pltpu.SemaphoreType.DMA, pltpu.SemaphoreType.DMA,
        pltpu.VMEM_SHARED((_ACC_R, _D), jnp.float32),
    ],
    compiler_params=pltpu.CompilerParams(use_tc_tiling_on_sc=False),
)(_prop_body)


def _mean_body(a, b, c, d, o):
    o[...] = (a[...] + b[...] + c[...] + d[...]) * 0.25


_mean4 = pl.pallas_call(
    _mean_body,
    grid=(_PN // 1600,),
    in_specs=[pl.BlockSpec((1600, _D), lambda i: (i, 0))] * 4,
    out_specs=pl.BlockSpec((1600, _D), lambda i: (i, 0)),
    out_shape=jax.ShapeDtypeStruct((_PN, _D), jnp.float32),
)


def kernel(user_emb, item_emb, edge_weight, edge_index):
    row = edge_index[0]
    col = edge_index[1]
    npad = _EARR - _E
    rowp = jnp.concatenate([row, jnp.full((npad,), -1, jnp.int32)])
    colp = jnp.concatenate(
        [col, (jnp.arange(npad, dtype=jnp.int32) * 7919) % (2 * _HALF)])
    wp = jnp.concatenate([edge_weight, jnp.zeros((npad,), jnp.float32)])
    rcw = jnp.stack([rowp, colp], axis=0)
    rcw = rcw.reshape(2, _EARR // _C, _C).transpose(1, 0, 2)
    zpad = jnp.zeros((_ACC_R - _HALF, _D), jnp.float32)
    ego0 = jnp.concatenate([user_emb, zpad, item_emb, zpad], axis=0)
    e1 = _propagate(ego0, rcw, wp)
    e2 = _propagate(e1, rcw, wp)
    e3 = _propagate(e2, rcw, wp)
    final = _mean4(ego0, e1, e2, e3)
    return final[:_N_USERS], final[_ACC_R:_ACC_R + _N_ITEMS]
